# Initial kernel scaffold; baseline (speedup 1.0000x reference)
import jax
import jax.numpy as jnp
from jax.experimental import pallas as pl


def kernel(trend_state, vol_state, liq_state, trend_w, vol_w, liq_w):
    raise NotImplementedError("write your pallas kernel here")


# speedup vs baseline: 4.5503x; 4.5503x over previous
import functools

import jax
import jax.numpy as jnp
from jax import lax
from jax.experimental import pallas as pl
from jax.experimental.pallas import tpu as pltpu
from jax.experimental.pallas import tpu_sc as plsc

B = 16384
EMB = 96
NS = 8
DIM = 32
FUSED = NS * NS * NS

NW = 32
BPW = B // NW
CHUNK = 128
NCH = BPW // CHUNK
L = 16


def _fuse_tables_body(tw_ref, vw_ref, lw_ref, out_ref):
    r = lax.broadcasted_iota(jnp.int32, (FUSED, NS), 0)
    c = lax.broadcasted_iota(jnp.int32, (FUSED, NS), 1)
    oh_t = ((r // 64) % NS == c).astype(jnp.float32)
    oh_v = ((r // 8) % NS == c).astype(jnp.float32)
    oh_l = (r % NS == c).astype(jnp.float32)
    t_big = jnp.dot(oh_t, tw_ref[...], preferred_element_type=jnp.float32)
    v_big = jnp.dot(oh_v, vw_ref[...], preferred_element_type=jnp.float32)
    l_big = jnp.dot(oh_l, lw_ref[...], preferred_element_type=jnp.float32)
    out_ref[...] = jnp.concatenate([t_big, v_big, l_big], axis=1)


_fuse_tables = pl.pallas_call(
    _fuse_tables_body,
    out_shape=jax.ShapeDtypeStruct((FUSED, EMB), jnp.float32),
)


@functools.lru_cache(maxsize=1)
def _make_sc_embed():
    mesh = plsc.VectorSubcoreMesh(core_axis_name="c", subcore_axis_name="s")

    @functools.partial(
        pl.kernel,
        out_type=jax.ShapeDtypeStruct((B, EMB), jnp.float32),
        mesh=mesh,
        scratch_types=[
            pltpu.VMEM((BPW,), jnp.int32),
            pltpu.VMEM((BPW,), jnp.int32),
            pltpu.VMEM((BPW,), jnp.int32),
            pltpu.VMEM((NCH, CHUNK), jnp.int32),
            pltpu.VMEM((BPW, EMB), jnp.float32),
            pltpu.SemaphoreType.DMA,
        ],
        compiler_params=pltpu.CompilerParams(use_tc_tiling_on_sc=False),
    )
    def _sc_embed(fused_hbm, t_hbm, v_hbm, l_hbm, out_hbm,
                  t_v, v_v, l_v, idx_v, rows_v, sem):
        wid = lax.axis_index("s") * 2 + lax.axis_index("c")
        base = wid * BPW

        pltpu.sync_copy(t_hbm.at[pl.ds(base, BPW)], t_v)
        pltpu.sync_copy(v_hbm.at[pl.ds(base, BPW)], v_v)
        pltpu.sync_copy(l_hbm.at[pl.ds(base, BPW)], l_v)

        for j in range(NCH):
            for i in range(CHUNK // L):
                off = j * CHUNK + i * L
                t = jnp.clip(t_v[pl.ds(off, L)], 0, NS - 1)
                v = jnp.clip(v_v[pl.ds(off, L)], 0, NS - 1)
                l = jnp.clip(l_v[pl.ds(off, L)], 0, NS - 1)
                idx_v[j, pl.ds(i * L, L)] = t * 64 + v * 8 + l

        handles = []
        for j in range(NCH):
            handles.append(
                pltpu.async_copy(
                    fused_hbm.at[idx_v.at[j]],
                    rows_v.at[pl.ds(j * CHUNK, CHUNK)],
                    sem,
                )
            )
        for h in handles:
            h.wait()

        pltpu.sync_copy(rows_v, out_hbm.at[pl.ds(base, BPW)])

    return _sc_embed


@jax.jit
def kernel(trend_state, vol_state, liq_state, trend_w, vol_w, liq_w):
    fused = _fuse_tables(trend_w, vol_w, liq_w)
    return _make_sc_embed()(fused, trend_state, vol_state, liq_state)


# speedup vs baseline: 5.3188x; 1.1689x over previous
import functools

import jax
import jax.numpy as jnp
from jax import lax
from jax.experimental import pallas as pl
from jax.experimental.pallas import tpu as pltpu
from jax.experimental.pallas import tpu_sc as plsc

B = 16384
EMB = 96
NS = 8
DIM = 32
FUSED = NS * NS * NS

NW = 32
BPW = B // NW
CHUNK = 128
NCH = BPW // CHUNK
L = 16


def _fuse_tables_body(tw_ref, vw_ref, lw_ref, out_ref):
    r = lax.broadcasted_iota(jnp.int32, (FUSED, NS), 0)
    c = lax.broadcasted_iota(jnp.int32, (FUSED, NS), 1)
    oh_t = ((r // 64) % NS == c).astype(jnp.float32)
    oh_v = ((r // 8) % NS == c).astype(jnp.float32)
    oh_l = (r % NS == c).astype(jnp.float32)
    t_big = jnp.dot(oh_t, tw_ref[...], preferred_element_type=jnp.float32)
    v_big = jnp.dot(oh_v, vw_ref[...], preferred_element_type=jnp.float32)
    l_big = jnp.dot(oh_l, lw_ref[...], preferred_element_type=jnp.float32)
    pad = jnp.zeros((FUSED, 128 - EMB), jnp.float32)
    out_ref[...] = jnp.concatenate([t_big, v_big, l_big, pad], axis=1)


_fuse_tables = pl.pallas_call(
    _fuse_tables_body,
    out_shape=jax.ShapeDtypeStruct((FUSED, 128), jnp.float32),
)


@functools.lru_cache(maxsize=1)
def _make_sc_embed():
    mesh = plsc.VectorSubcoreMesh(core_axis_name="c", subcore_axis_name="s")

    @functools.partial(
        pl.kernel,
        out_type=jax.ShapeDtypeStruct((B, 128), jnp.float32),
        mesh=mesh,
        scratch_types=[
            pltpu.VMEM((BPW,), jnp.int32),
            pltpu.VMEM((BPW,), jnp.int32),
            pltpu.VMEM((BPW,), jnp.int32),
            pltpu.VMEM((NCH, CHUNK), jnp.int32),
            pltpu.VMEM((BPW, 128), jnp.float32),
            pltpu.SemaphoreType.DMA,
        ],
        compiler_params=pltpu.CompilerParams(use_tc_tiling_on_sc=True),
    )
    def _sc_embed(fused_hbm, t_hbm, v_hbm, l_hbm, out_hbm,
                  t_v, v_v, l_v, idx_v, rows_v, sem):
        wid = lax.axis_index("s") * 2 + lax.axis_index("c")
        base = wid * BPW

        pltpu.sync_copy(t_hbm.at[pl.ds(base, BPW)], t_v)
        pltpu.sync_copy(v_hbm.at[pl.ds(base, BPW)], v_v)
        pltpu.sync_copy(l_hbm.at[pl.ds(base, BPW)], l_v)

        for j in range(NCH):
            for i in range(CHUNK // L):
                off = j * CHUNK + i * L
                t = jnp.clip(t_v[pl.ds(off, L)], 0, NS - 1)
                v = jnp.clip(v_v[pl.ds(off, L)], 0, NS - 1)
                l = jnp.clip(l_v[pl.ds(off, L)], 0, NS - 1)
                idx_v[j, pl.ds(i * L, L)] = t * 64 + v * 8 + l

        handles = []
        for j in range(NCH):
            handles.append(
                pltpu.async_copy(
                    fused_hbm.at[idx_v.at[j]],
                    rows_v.at[pl.ds(j * CHUNK, CHUNK)],
                    sem,
                )
            )
        for h in handles:
            h.wait()

        pltpu.sync_copy(rows_v, out_hbm.at[pl.ds(base, BPW)])

    return _sc_embed


@jax.jit
def kernel(trend_state, vol_state, liq_state, trend_w, vol_w, liq_w):
    fused = _fuse_tables(trend_w, vol_w, liq_w)
    padded = _make_sc_embed()(fused, trend_state, vol_state, liq_state)
    return padded[:, :EMB]
